# baseline (speedup 1.0000x reference)
import jax
import jax.numpy as jnp
from jax import lax
from jax.experimental import pallas as pl
from jax.experimental.pallas import tpu as pltpu
from jax.experimental.pallas import tpu_sc as plsc

K = 8192
C = 256
NTOK = 16384
MT = 2048
NCH = 1024
NWORK = 32
TPW = NTOK // NWORK
GCH = 256
CW = 16


def _nn_body(z_ref, cb_ref, w_ref, b_ref, idx_ref, mind_ref, cnt_ref,
             pcb_ref, en_ref):
    @pl.when(pl.program_id(0) == 0)
    def _project():
        pc = jnp.dot(cb_ref[...], w_ref[...],
                     preferred_element_type=jnp.float32) + b_ref[...]
        pcb_ref[...] = pc
        ones = jnp.ones((1, C), jnp.float32)
        en_ref[...] = lax.dot_general(
            ones, pc * pc, (((1,), (1,)), ((), ())),
            precision=lax.Precision.HIGHEST,
            preferred_element_type=jnp.float32)

    zt = z_ref[...]
    z2 = jnp.sum(zt * zt, axis=1, keepdims=True)
    ztm2 = zt * (-2.0)

    def body(n, carry):
        bv, bi = carry
        pc = pcb_ref[pl.ds(n * NCH, NCH), :]
        dots2 = lax.dot_general(
            ztm2, pc, (((1,), (1,)), ((), ())),
            preferred_element_type=jnp.float32)
        emb = en_ref[0:1, pl.ds(n * NCH, NCH)]
        dists = (z2 + emb) + dots2
        mv = jnp.min(dists, axis=1, keepdims=True)
        mi = (jnp.argmin(dists, axis=1).astype(jnp.int32)
              .reshape(MT, 1) + n * NCH)
        upd = mv < bv
        return jnp.where(upd, mv, bv), jnp.where(upd, mi, bi)

    bv0 = jnp.full((MT, 1), jnp.inf, jnp.float32)
    bi0 = jnp.zeros((MT, 1), jnp.int32)
    bv, bi = lax.fori_loop(0, K // NCH, body, (bv0, bi0))
    idx_ref[...] = bi
    mind_ref[...] = bv
    hi = bi // NCH
    lo = bi - hi * NCH
    ioh = lax.broadcasted_iota(jnp.int32, (MT, K // NCH), 1)
    iol = lax.broadcasted_iota(jnp.int32, (MT, NCH), 1)
    ohh = jnp.where(hi == ioh, 1.0, 0.0)
    ohl = jnp.where(lo == iol, 1.0, 0.0)
    ct = lax.dot_general(ohh, ohl, (((0,), (0,)), ((), ())),
                         preferred_element_type=jnp.float32)
    cnt_ref[...] = ct.reshape(1, K // NCH, NCH)


def _stats_body(c2_ref, mind_ref, counts_ref, avg_ref, loss_ref,
                ppl_ref, usage_ref, tot_ref):
    c = jnp.sum(c2_ref[...], axis=0)
    counts_ref[...] = c
    tot = jnp.maximum(jnp.sum(c), 1.0)
    tot_ref[...] = jnp.reshape(tot, (1, 1))
    avg = c / tot
    avg_ref[...] = avg
    safe = jnp.where(avg > 0, avg, 1.0)
    ent = jnp.sum(avg * jnp.log(safe + 1e-10))
    ppl_ref[...] = jnp.reshape(jnp.exp(-ent), (1, 1))
    usage_ref[...] = jnp.reshape(
        jnp.sum((c > 0).astype(jnp.float32)) * (1.0 / K), (1, 1))
    m = jnp.sum(mind_ref[...]) * (1.0 / (NTOK * C))
    loss_ref[...] = jnp.reshape(0.25 * m + m, (1, 1))


def _sc_body(pcb_hbm, idx_hbm, zq_hbm, idx_v0, idx_v1, rows_v, sem):
    cc = lax.axis_index("c")
    ss = lax.axis_index("s")
    wid = ss * 2 + cc
    base = wid * TPW
    pltpu.sync_copy(idx_hbm.at[pl.ds(base, GCH)], idx_v0)
    pltpu.sync_copy(idx_hbm.at[pl.ds(base + GCH, GCH)], idx_v1)
    pltpu.async_copy(pcb_hbm.at[idx_v0], rows_v, sem).wait()
    pltpu.sync_copy(rows_v, zq_hbm.at[pl.ds(base, GCH)])
    pltpu.async_copy(pcb_hbm.at[idx_v1], rows_v, sem).wait()
    pltpu.sync_copy(rows_v, zq_hbm.at[pl.ds(base + GCH, GCH)])


def kernel(z, codebook, W, proj_bias):
    B, T, Cz = z.shape
    zf = z.reshape(B * T, Cz)

    nmt = NTOK // MT
    idx2, mind2, cnt_part, pcb, _en = pl.pallas_call(
        _nn_body,
        grid=(nmt,),
        in_specs=[pl.BlockSpec((MT, C), lambda m: (m, 0)),
                  pl.BlockSpec((K, C), lambda m: (0, 0)),
                  pl.BlockSpec((C, C), lambda m: (0, 0)),
                  pl.BlockSpec((1, C), lambda m: (0, 0))],
        out_specs=[pl.BlockSpec((MT, 1), lambda m: (m, 0)),
                   pl.BlockSpec((MT, 1), lambda m: (m, 0)),
                   pl.BlockSpec((1, K // NCH, NCH), lambda m: (m, 0, 0)),
                   pl.BlockSpec((K, C), lambda m: (0, 0)),
                   pl.BlockSpec((1, K), lambda m: (0, 0))],
        out_shape=[jax.ShapeDtypeStruct((NTOK, 1), jnp.int32),
                   jax.ShapeDtypeStruct((NTOK, 1), jnp.float32),
                   jax.ShapeDtypeStruct((NTOK // MT, K // NCH, NCH),
                                        jnp.float32),
                   jax.ShapeDtypeStruct((K, C), jnp.float32),
                   jax.ShapeDtypeStruct((1, K), jnp.float32)],
    )(zf, codebook, W, proj_bias.reshape(1, C))

    idx_flat = idx2.reshape(NTOK)
    mesh = plsc.VectorSubcoreMesh(core_axis_name="c", subcore_axis_name="s")
    zq_flat = pl.kernel(
        _sc_body,
        out_type=jax.ShapeDtypeStruct((NTOK, C), jnp.float32),
        mesh=mesh,
        scratch_types=[
            pltpu.VMEM((GCH,), jnp.int32),
            pltpu.VMEM((GCH,), jnp.int32),
            pltpu.VMEM((GCH, C), jnp.float32),
            pltpu.SemaphoreType.DMA,
        ],
    )(pcb, idx_flat)

    counts2, avg2, loss, ppl, usage, tot = pl.pallas_call(
        _stats_body,
        out_shape=[jax.ShapeDtypeStruct((K // NCH, NCH), jnp.float32),
                   jax.ShapeDtypeStruct((K // NCH, NCH), jnp.float32),
                   jax.ShapeDtypeStruct((1, 1), jnp.float32),
                   jax.ShapeDtypeStruct((1, 1), jnp.float32),
                   jax.ShapeDtypeStruct((1, 1), jnp.float32),
                   jax.ShapeDtypeStruct((1, 1), jnp.float32)],
    )(cnt_part, mind2)

    return (zq_flat.reshape(B, T, Cz),
            loss.reshape(()),
            ppl.reshape(()),
            avg2.reshape(K),
            usage.reshape(()),
            counts2.reshape(K),
            tot.reshape(()),
            idx2.reshape(B, T))


# speedup vs baseline: 1.6063x; 1.6063x over previous
import jax
import jax.numpy as jnp
from jax import lax
from jax.experimental import pallas as pl
from jax.experimental.pallas import tpu as pltpu
from jax.experimental.pallas import tpu_sc as plsc

K = 8192
C = 256
NTOK = 16384
MT = 4096
NCH = 1024
NWORK = 32
TPW = NTOK // NWORK
GCH = 256
CW = 16


def _nn_body(z_ref, cb_ref, w_ref, b_ref, idx_ref, mind_ref, cnt_ref,
             pcb_ref, en_ref):
    @pl.when(pl.program_id(0) == 0)
    def _project():
        pc = jnp.dot(cb_ref[...], w_ref[...],
                     preferred_element_type=jnp.float32) + b_ref[...]
        pcb_ref[...] = pc
        ones = jnp.ones((1, C), jnp.float32)
        en_ref[...] = lax.dot_general(
            ones, pc * pc, (((1,), (1,)), ((), ())),
            precision=lax.Precision.HIGHEST,
            preferred_element_type=jnp.float32)

    zt = z_ref[...]
    z2 = jnp.sum(zt * zt, axis=1, keepdims=True)
    ztm2 = zt * (-2.0)

    def body(n, carry):
        bv, bi = carry
        pc = pcb_ref[pl.ds(n * NCH, NCH), :]
        dots2 = lax.dot_general(
            ztm2, pc, (((1,), (1,)), ((), ())),
            preferred_element_type=jnp.float32)
        emb = en_ref[0:1, pl.ds(n * NCH, NCH)]
        dists = emb + dots2
        mv = jnp.min(dists, axis=1, keepdims=True)
        iota = lax.broadcasted_iota(jnp.int32, dists.shape, 1)
        cand = jnp.where(dists == mv, iota, jnp.int32(K))
        mi = jnp.min(cand, axis=1, keepdims=True) + n * NCH
        upd = mv < bv
        return jnp.where(upd, mv, bv), jnp.where(upd, mi, bi)

    bv0 = jnp.full((MT, 1), jnp.inf, jnp.float32)
    bi0 = jnp.zeros((MT, 1), jnp.int32)
    bv, bi = lax.fori_loop(0, K // NCH, body, (bv0, bi0))
    idx_ref[...] = bi
    mind_ref[...] = z2 + bv
    hi = bi // NCH
    lo = bi - hi * NCH
    ioh = lax.broadcasted_iota(jnp.int32, (MT, K // NCH), 1)
    iol = lax.broadcasted_iota(jnp.int32, (MT, NCH), 1)
    ohh = jnp.where(hi == ioh, 1.0, 0.0)
    ohl = jnp.where(lo == iol, 1.0, 0.0)
    ct = lax.dot_general(ohh, ohl, (((0,), (0,)), ((), ())),
                         preferred_element_type=jnp.float32)
    cnt_ref[...] = ct.reshape(1, K // NCH, NCH)


def _stats_body(c2_ref, mind_ref, counts_ref, avg_ref, loss_ref,
                ppl_ref, usage_ref, tot_ref):
    c = jnp.sum(c2_ref[...], axis=0)
    counts_ref[...] = c
    tot = jnp.maximum(jnp.sum(c), 1.0)
    tot_ref[...] = jnp.reshape(tot, (1, 1))
    avg = c / tot
    avg_ref[...] = avg
    safe = jnp.where(avg > 0, avg, 1.0)
    ent = jnp.sum(avg * jnp.log(safe + 1e-10))
    ppl_ref[...] = jnp.reshape(jnp.exp(-ent), (1, 1))
    usage_ref[...] = jnp.reshape(
        jnp.sum((c > 0).astype(jnp.float32)) * (1.0 / K), (1, 1))
    m = jnp.sum(mind_ref[...]) * (1.0 / (NTOK * C))
    loss_ref[...] = jnp.reshape(0.25 * m + m, (1, 1))


def _sc_body(pcb_hbm, idx_hbm, zq_hbm, idx_v0, idx_v1, rows_v, sem):
    cc = lax.axis_index("c")
    ss = lax.axis_index("s")
    wid = ss * 2 + cc
    base = wid * TPW
    pltpu.sync_copy(idx_hbm.at[pl.ds(base, GCH)], idx_v0)
    pltpu.sync_copy(idx_hbm.at[pl.ds(base + GCH, GCH)], idx_v1)
    pltpu.async_copy(pcb_hbm.at[idx_v0], rows_v, sem).wait()
    pltpu.sync_copy(rows_v, zq_hbm.at[pl.ds(base, GCH)])
    pltpu.async_copy(pcb_hbm.at[idx_v1], rows_v, sem).wait()
    pltpu.sync_copy(rows_v, zq_hbm.at[pl.ds(base + GCH, GCH)])


def kernel(z, codebook, W, proj_bias):
    B, T, Cz = z.shape
    zf = z.reshape(B * T, Cz)

    nmt = NTOK // MT
    idx2, mind2, cnt_part, pcb, _en = pl.pallas_call(
        _nn_body,
        grid=(nmt,),
        in_specs=[pl.BlockSpec((MT, C), lambda m: (m, 0)),
                  pl.BlockSpec((K, C), lambda m: (0, 0)),
                  pl.BlockSpec((C, C), lambda m: (0, 0)),
                  pl.BlockSpec((1, C), lambda m: (0, 0))],
        out_specs=[pl.BlockSpec((MT, 1), lambda m: (m, 0)),
                   pl.BlockSpec((MT, 1), lambda m: (m, 0)),
                   pl.BlockSpec((1, K // NCH, NCH), lambda m: (m, 0, 0)),
                   pl.BlockSpec((K, C), lambda m: (0, 0)),
                   pl.BlockSpec((1, K), lambda m: (0, 0))],
        out_shape=[jax.ShapeDtypeStruct((NTOK, 1), jnp.int32),
                   jax.ShapeDtypeStruct((NTOK, 1), jnp.float32),
                   jax.ShapeDtypeStruct((NTOK // MT, K // NCH, NCH),
                                        jnp.float32),
                   jax.ShapeDtypeStruct((K, C), jnp.float32),
                   jax.ShapeDtypeStruct((1, K), jnp.float32)],
    )(zf, codebook, W, proj_bias.reshape(1, C))

    idx_flat = idx2.reshape(NTOK)
    mesh = plsc.VectorSubcoreMesh(core_axis_name="c", subcore_axis_name="s")
    zq_flat = pl.kernel(
        _sc_body,
        out_type=jax.ShapeDtypeStruct((NTOK, C), jnp.float32),
        mesh=mesh,
        scratch_types=[
            pltpu.VMEM((GCH,), jnp.int32),
            pltpu.VMEM((GCH,), jnp.int32),
            pltpu.VMEM((GCH, C), jnp.float32),
            pltpu.SemaphoreType.DMA,
        ],
    )(pcb, idx_flat)

    counts2, avg2, loss, ppl, usage, tot = pl.pallas_call(
        _stats_body,
        out_shape=[jax.ShapeDtypeStruct((K // NCH, NCH), jnp.float32),
                   jax.ShapeDtypeStruct((K // NCH, NCH), jnp.float32),
                   jax.ShapeDtypeStruct((1, 1), jnp.float32),
                   jax.ShapeDtypeStruct((1, 1), jnp.float32),
                   jax.ShapeDtypeStruct((1, 1), jnp.float32),
                   jax.ShapeDtypeStruct((1, 1), jnp.float32)],
    )(cnt_part, mind2)

    return (zq_flat.reshape(B, T, Cz),
            loss.reshape(()),
            ppl.reshape(()),
            avg2.reshape(K),
            usage.reshape(()),
            counts2.reshape(K),
            tot.reshape(()),
            idx2.reshape(B, T))
